# baseline (speedup 1.0000x reference)
import functools

import jax
import jax.numpy as jnp
from jax import lax
from jax.experimental import pallas as pl
from jax.experimental.pallas import tpu as pltpu
from jax.experimental.pallas import tpu_sc as plsc

SB = 256
FW = 256
S = 8192
F = 2048
NG = FW // 16


def _sc_cumsum(x_hbm, out_hbm, buf):
    c = lax.axis_index("c")
    s = lax.axis_index("s")
    wid = s * 2 + c
    b = wid // (F // FW)
    f0 = (wid % (F // FW)) * FW

    def chunk_body(cs, carry):
        row0 = cs * SB
        pltpu.sync_copy(x_hbm.at[b, pl.ds(row0, SB), pl.ds(f0, FW)], buf)

        def row_body(r, carry):
            new = []
            for g in range(NG):
                v = buf[r, pl.ds(g * 16, 16)] + carry[g]
                buf[r, pl.ds(g * 16, 16)] = v
                new.append(v)
            return tuple(new)

        carry = lax.fori_loop(0, SB, row_body, carry)
        pltpu.sync_copy(buf, out_hbm.at[b, pl.ds(row0, SB), pl.ds(f0, FW)])
        return carry

    init = tuple(jnp.zeros((16,), jnp.float32) for _ in range(NG))
    lax.fori_loop(0, S // SB, chunk_body, init)


def kernel(x):
    mesh = plsc.VectorSubcoreMesh(core_axis_name="c", subcore_axis_name="s")
    f = pl.kernel(
        _sc_cumsum,
        mesh=mesh,
        out_type=jax.ShapeDtypeStruct(x.shape, x.dtype),
        scratch_types=[pltpu.VMEM((SB, FW), jnp.float32)],
    )
    return f(x)
